# baseline (speedup 1.0000x reference)
import functools

import jax
import jax.numpy as jnp
from jax import lax
from jax.experimental import pallas as pl
from jax.experimental.pallas import tpu as pltpu
from jax.experimental.pallas import tpu_sc as plsc

SEQ = 200
DIM = 64
LANES = 128
NWORKERS = 32
NBUF = 4


@functools.lru_cache(maxsize=None)
def _build(batch):
    bcols = batch // LANES
    mesh = plsc.VectorSubcoreMesh(core_axis_name="c", subcore_axis_name="s")
    info = plsc.get_sparse_core_info()
    nc = info.num_cores

    @functools.partial(
        pl.kernel,
        out_type=jax.ShapeDtypeStruct((SEQ, DIM, batch), jnp.float32),
        mesh=mesh,
        scratch_types=[
            pltpu.VMEM((NBUF, LANES), jnp.int32),
            pltpu.VMEM((NBUF, LANES, LANES), jnp.float32),
            pltpu.VMEM((2, DIM, LANES + 1), jnp.float32),
            pltpu.VMEM((NBUF, DIM), jnp.float32),
            [pltpu.SemaphoreType.DMA] * NBUF,
            [pltpu.SemaphoreType.DMA] * NBUF,
            [pltpu.SemaphoreType.DMA] * 2,
        ],
        compiler_params=pltpu.CompilerParams(use_tc_tiling_on_sc=True,
                                             needs_layout_passes=False),
    )
    def emb(idx_hbm, tok_hbm, pos_hbm, out_hbm, idx_v, rows_v, out_v,
            pos_v, si, sg, sw):
        w = lax.axis_index("s") * nc + lax.axis_index("c")
        lane = lax.iota(jnp.int32, 16)

        def fire_idx(s, b):
            pltpu.async_copy(idx_hbm.at[s // 8, w, s % 8], idx_v.at[b], si[b])
            pltpu.async_copy(pos_hbm.at[pl.ds(s * DIM, DIM)], pos_v.at[b],
                             si[b])

        def wait_idx(b):
            pltpu.make_async_copy(idx_hbm.at[0, w, 0], idx_v.at[b],
                                  si[b]).wait()
            pltpu.make_async_copy(pos_hbm.at[pl.ds(0, DIM)], pos_v.at[b],
                                  si[b]).wait()

        def fire_gather(b):
            for q in range(4):
                pltpu.async_copy(tok_hbm.at[idx_v.at[b].at[pl.ds(q * 32, 32)]],
                                 rows_v.at[b].at[pl.ds(q * 32, 32)], sg[b])

        def drain_gather(b):
            for q in range(4):
                pltpu.make_async_copy(
                    tok_hbm.at[idx_v.at[b].at[pl.ds(q * 32, 32)]],
                    rows_v.at[b].at[pl.ds(q * 32, 32)], sg[b]).wait()

        def compute(s, b):
            pvecs = tuple(pos_v[b, pl.ds(dg * 16, 16)]
                          for dg in range(DIM // 16))

            ob = b % 2

            def brow(bi, pv):
                bsp = jnp.broadcast_to(bi, (16,))
                for dg in range(DIM // 16):
                    v = rows_v[b, bi, pl.ds(dg * 16, 16)] + pv[dg]
                    plsc.store_scatter(out_v.at[ob],
                                       [lane + (dg * 16), bsp], v)
                return pv

            lax.fori_loop(0, LANES, brow, pvecs, unroll=4)

        def fire_wb(s, ob):
            pltpu.async_copy(out_v.at[ob].at[:, pl.ds(0, LANES)],
                             out_hbm.at[s, :, pl.ds(w * LANES, LANES)], sw[ob])

        def wait_wb(ob):
            pltpu.make_async_copy(out_v.at[ob].at[:, pl.ds(0, LANES)],
                                  out_hbm.at[0, :, pl.ds(w * LANES, LANES)],
                                  sw[ob]).wait()

        for b in range(NBUF - 1):
            fire_idx(b, b)
        for b in range(NBUF - 2):
            wait_idx(b)
            fire_gather(b)

        def body(k, carry):
            for b in range(NBUF):
                s = NBUF * k + b

                @pl.when(s + NBUF - 1 < SEQ)
                def _prefetch_idx():
                    fire_idx(s + NBUF - 1, (b + NBUF - 1) % NBUF)

                @pl.when(s + NBUF - 2 < SEQ)
                def _start_gather():
                    wait_idx((b + NBUF - 2) % NBUF)
                    fire_gather((b + NBUF - 2) % NBUF)

                @pl.when(s >= 2)
                def _reclaim_out():
                    wait_wb(b % 2)

                drain_gather(b)
                compute(s, b)
                fire_wb(s, b % 2)
            return carry

        lax.fori_loop(0, SEQ // NBUF, body, 0)
        for ob in range(2):
            wait_wb(ob)

    return emb


def kernel(inputs, token_table, position_table):
    batch, seq = inputs.shape
    dim = token_table.shape[1]
    idx4 = (inputs.astype(jnp.int32).T
            .reshape(seq // 8, 8, batch // LANES, LANES)
            .transpose(0, 2, 1, 3))
    tok_p = jnp.pad(token_table, ((0, 0), (0, LANES - dim)))
    pos_flat = position_table.reshape(seq * dim)
    outt = _build(batch)(idx4, tok_p, pos_flat)
    return outt.transpose(2, 0, 1)
